# baseline (speedup 1.0000x reference)
import jax
import jax.numpy as jnp
from jax.experimental import pallas as pl


def _router_body(x_ref, w_ref, o_ref):
    logits = jnp.dot(x_ref[...], w_ref[...], preferred_element_type=jnp.float32)
    m = jnp.max(logits, axis=-1, keepdims=True)
    e = jnp.exp(logits - m)
    o_ref[...] = e / jnp.sum(e, axis=-1, keepdims=True)


def kernel(x, expert_embeddings):
    B, S, H, D = x.shape
    E = expert_embeddings.shape[1]
    R = B * S * H
    x2 = x.reshape(R, D)
    BLK = 2048
    out = pl.pallas_call(
        _router_body,
        grid=(R // BLK,),
        in_specs=[
            pl.BlockSpec((BLK, D), lambda i: (i, 0)),
            pl.BlockSpec((D, E), lambda i: (0, 0)),
        ],
        out_specs=pl.BlockSpec((BLK, E), lambda i: (i, 0)),
        out_shape=jax.ShapeDtypeStruct((R, E), jnp.float32),
    )(x2, expert_embeddings)
    return out.reshape(B, S, H, E)
